# Initial kernel scaffold; baseline (speedup 1.0000x reference)
import jax
import jax.numpy as jnp
from jax.experimental import pallas as pl


def kernel(sequence, params):
    raise NotImplementedError("write your pallas kernel here")


# speedup vs baseline: 5.9430x; 5.9430x over previous
import jax
import jax.numpy as jnp
from jax.experimental import pallas as pl
from jax.experimental.pallas import tpu as pltpu

SEQ_LEN = 131072
NUM_ANCHORS = 5
PRE_N = 6000
POST_N = 300
NMS_THRESH = 0.7
NPAD = 6016
OUT_R = 512
BASE_ANCHORS = jnp.array([[-4.0, 3.0], [-8.0, 7.0], [-16.0, 15.0], [-32.0, 31.0], [-64.0, 63.0]], dtype=jnp.float32)

ENC_SPEC = [(14, 32, 3, 1, 1, 16), (32, 16, 3, 1, 1, 8), (16, 8, 3, 2, 2, 4), (8, 4, 3, 2, 2, 2), (4, 2, 3, 3, 3, 1)]
DEC_SPEC = [(2, 4, 3, 3, 3, 2), (8, 8, 3, 2, 2, 4), (16, 16, 3, 2, 2, 8), (32, 32, 3, 1, 1, 16), (64, 32, 3, 1, 1, 16)]


def _conv1d(x, w, b=None, pad=0, dil=1, groups=1):
    y = jax.lax.conv_general_dilated(x, w, window_strides=(1,), padding=[(pad, pad)], rhs_dilation=(dil,), dimension_numbers=('NCH', 'OIH', 'NCH'), feature_group_count=groups)
    if b is not None:
        y = y + b[None, :, None]
    return y


def _batchnorm(x, g, b, eps=1e-5):
    m = x.mean(axis=(0, 2), keepdims=True)
    v = ((x - m) ** 2).mean(axis=(0, 2), keepdims=True)
    return g[None, :, None] * (x - m) / jnp.sqrt(v + eps) + b[None, :, None]


def _ads_conv(x, p, pad, dil):
    C = x.shape[1]
    h = _conv1d(x, p['dw_w'], p['dw_b'], pad=pad, dil=dil, groups=C)
    h = jax.nn.relu(h)
    ak = p['attn_w'].shape[-1]
    a = _conv1d(h, p['attn_w'], p['attn_b'], pad=(ak - 1) // 2, dil=1, groups=C)
    h = h * jax.nn.sigmoid(a)
    s = h.mean(axis=2)
    s = jax.nn.relu(s @ p['se_w1'].T + p['se_b1'])
    s = jax.nn.sigmoid(s @ p['se_w2'].T + p['se_b2'])
    h = h * s[:, :, None]
    return _conv1d(h, p['pw_w'], p['pw_b'])


def _backbone(sequence, params):
    L = sequence.shape[-1]
    out = sequence
    inter = []
    for p, (cin, cout, k, pad, dil, rr) in zip(params['enc'], ENC_SPEC):
        out = _batchnorm(jax.nn.relu(_ads_conv(out, p, pad, dil)), p['bn_g'], p['bn_b'])
        inter.append(out)
    inter.pop()
    for p, (cin, cout, k, pad, dil, rr) in zip(params['dec'][:-1], DEC_SPEC[:-1]):
        out = _batchnorm(jax.nn.relu(_ads_conv(out, p, pad, dil)), p['bn_g'], p['bn_b'])
        out = jnp.concatenate([out, inter.pop()], axis=1)
    p = params['dec'][-1]
    cin, cout, k, pad, dil, rr = DEC_SPEC[-1]
    feat = _batchnorm(jax.nn.relu(_ads_conv(out, p, pad, dil)), p['bn_g'], p['bn_b'])

    rp = params['rpn']
    r = _conv1d(feat, rp['dw_w'], rp['dw_b'], pad=1, dil=1, groups=32)
    r = _conv1d(r, rp['pw_w'], rp['pw_b'])
    r = _batchnorm(jax.nn.relu(r), rp['bn_g'], rp['bn_b'])

    cls = _conv1d(r, params['cls_w'], params['cls_b'])
    prob = jax.nn.sigmoid(cls).transpose(0, 2, 1)
    box = _conv1d(r, params['box_w'], params['box_b']).transpose(0, 2, 1)

    scores = prob.reshape(-1)
    deltas = box.reshape(-1, 2)
    shifts = jnp.arange(L, dtype=jnp.float32)
    anc = (shifts[:, None, None] + BASE_ANCHORS[None, :, :]).reshape(-1, 2)
    w = anc[:, 1] - anc[:, 0] + 1.0
    ctr = anc[:, 0] + 0.5 * w
    pred_ctr = deltas[:, 0] * w + ctr
    pred_w = jnp.exp(jnp.clip(deltas[:, 1], -10.0, 10.0)) * w
    s = jnp.clip(pred_ctr - 0.5 * pred_w, 0.0, L - 1.0)
    e = jnp.clip(pred_ctr + 0.5 * pred_w, 0.0, L - 1.0)
    return scores, s, e


def _nms_body(scr, ssr, eer, scc, ssc, eec, osc, oss, oee, sup_ref):
    osc[...] = jnp.zeros_like(osc)
    oss[...] = jnp.zeros_like(oss)
    oee[...] = jnp.zeros_like(oee)
    sup_ref[...] = jnp.zeros_like(sup_ref)
    s_row = ssr[0:1, :]
    e_row = eer[0:1, :]
    lens = e_row - s_row + 1.0
    lane = jax.lax.broadcasted_iota(jnp.int32, (1, NPAD), 1)

    def body(i, cursor):
        sup = sup_ref[0:1, :] != 0.0
        si = ssc[pl.ds(i, 1), :]
        ei = eec[pl.ds(i, 1), :]
        sup_i = jnp.any(jnp.logical_and(sup, lane == i))
        li = ei - si + 1.0
        inter = jnp.maximum(0.0, jnp.minimum(ei, e_row) - jnp.maximum(si, s_row) + 1.0)
        iou = inter / (li + lens - inter)
        newly = (iou > NMS_THRESH) & (lane > i) & jnp.logical_not(sup_i)
        keep_i = jnp.logical_not(sup_i)
        sup_ref[0:1, :] = jnp.where(sup | newly, 1.0, 0.0)

        @pl.when(keep_i & (cursor < POST_N))
        def _():
            osc[pl.ds(cursor, 1), :] = scc[pl.ds(i, 1), :]
            oss[pl.ds(cursor, 1), :] = si
            oee[pl.ds(cursor, 1), :] = ei

        return cursor + keep_i.astype(jnp.int32)

    jax.lax.fori_loop(0, PRE_N, body, jnp.int32(0))


def _nms_topk(sc, ss, ee):
    pad = NPAD - PRE_N
    neg = jnp.full((pad,), -3.0e9, jnp.float32)
    scr = jnp.concatenate([sc, jnp.zeros((pad,), jnp.float32)]).reshape(1, NPAD)
    ssr = jnp.concatenate([ss, neg]).reshape(1, NPAD)
    eer = jnp.concatenate([ee, neg]).reshape(1, NPAD)
    scc = scr.reshape(NPAD, 1)
    ssc = ssr.reshape(NPAD, 1)
    eec = eer.reshape(NPAD, 1)
    out = pl.pallas_call(
        _nms_body,
        out_shape=[jax.ShapeDtypeStruct((OUT_R, 1), jnp.float32)] * 3,
        scratch_shapes=[pltpu.VMEM((1, NPAD), jnp.float32)],
    )(scr, ssr, eer, scc, ssc, eec)
    osc, oss, oee = out
    return jnp.stack([osc[:POST_N, 0], oss[:POST_N, 0], oee[:POST_N, 0]], axis=-1)


def kernel(sequence, params):
    scores, s, e = _backbone(sequence, params)
    topv, topi = jax.lax.top_k(scores, PRE_N)
    ss = s[topi]
    ee = e[topi]
    out3 = _nms_topk(topv, ss, ee)
    return out3[:, None, :]


# speedup vs baseline: 7.2489x; 1.2197x over previous
import jax
import jax.numpy as jnp
from jax.experimental import pallas as pl
from jax.experimental.pallas import tpu as pltpu

SEQ_LEN = 131072
NUM_ANCHORS = 5
PRE_N = 6000
POST_N = 300
NMS_THRESH = 0.7
NPAD = 6016
OUT_R = 512
BASE_ANCHORS = jnp.array([[-4.0, 3.0], [-8.0, 7.0], [-16.0, 15.0], [-32.0, 31.0], [-64.0, 63.0]], dtype=jnp.float32)

ENC_SPEC = [(14, 32, 3, 1, 1, 16), (32, 16, 3, 1, 1, 8), (16, 8, 3, 2, 2, 4), (8, 4, 3, 2, 2, 2), (4, 2, 3, 3, 3, 1)]
DEC_SPEC = [(2, 4, 3, 3, 3, 2), (8, 8, 3, 2, 2, 4), (16, 16, 3, 2, 2, 8), (32, 32, 3, 1, 1, 16), (64, 32, 3, 1, 1, 16)]


def _conv1d(x, w, b=None, pad=0, dil=1, groups=1):
    y = jax.lax.conv_general_dilated(x, w, window_strides=(1,), padding=[(pad, pad)], rhs_dilation=(dil,), dimension_numbers=('NCH', 'OIH', 'NCH'), feature_group_count=groups)
    if b is not None:
        y = y + b[None, :, None]
    return y


def _batchnorm(x, g, b, eps=1e-5):
    m = x.mean(axis=(0, 2), keepdims=True)
    v = ((x - m) ** 2).mean(axis=(0, 2), keepdims=True)
    return g[None, :, None] * (x - m) / jnp.sqrt(v + eps) + b[None, :, None]


def _ads_conv(x, p, pad, dil):
    C = x.shape[1]
    h = _conv1d(x, p['dw_w'], p['dw_b'], pad=pad, dil=dil, groups=C)
    h = jax.nn.relu(h)
    ak = p['attn_w'].shape[-1]
    a = _conv1d(h, p['attn_w'], p['attn_b'], pad=(ak - 1) // 2, dil=1, groups=C)
    h = h * jax.nn.sigmoid(a)
    s = h.mean(axis=2)
    s = jax.nn.relu(s @ p['se_w1'].T + p['se_b1'])
    s = jax.nn.sigmoid(s @ p['se_w2'].T + p['se_b2'])
    h = h * s[:, :, None]
    return _conv1d(h, p['pw_w'], p['pw_b'])


def _backbone(sequence, params):
    L = sequence.shape[-1]
    out = sequence
    inter = []
    for p, (cin, cout, k, pad, dil, rr) in zip(params['enc'], ENC_SPEC):
        out = _batchnorm(jax.nn.relu(_ads_conv(out, p, pad, dil)), p['bn_g'], p['bn_b'])
        inter.append(out)
    inter.pop()
    for p, (cin, cout, k, pad, dil, rr) in zip(params['dec'][:-1], DEC_SPEC[:-1]):
        out = _batchnorm(jax.nn.relu(_ads_conv(out, p, pad, dil)), p['bn_g'], p['bn_b'])
        out = jnp.concatenate([out, inter.pop()], axis=1)
    p = params['dec'][-1]
    cin, cout, k, pad, dil, rr = DEC_SPEC[-1]
    feat = _batchnorm(jax.nn.relu(_ads_conv(out, p, pad, dil)), p['bn_g'], p['bn_b'])

    rp = params['rpn']
    r = _conv1d(feat, rp['dw_w'], rp['dw_b'], pad=1, dil=1, groups=32)
    r = _conv1d(r, rp['pw_w'], rp['pw_b'])
    r = _batchnorm(jax.nn.relu(r), rp['bn_g'], rp['bn_b'])

    cls = _conv1d(r, params['cls_w'], params['cls_b'])
    prob = jax.nn.sigmoid(cls).transpose(0, 2, 1)
    box = _conv1d(r, params['box_w'], params['box_b']).transpose(0, 2, 1)

    scores = prob.reshape(-1)
    deltas = box.reshape(-1, 2)
    shifts = jnp.arange(L, dtype=jnp.float32)
    anc = (shifts[:, None, None] + BASE_ANCHORS[None, :, :]).reshape(-1, 2)
    w = anc[:, 1] - anc[:, 0] + 1.0
    ctr = anc[:, 0] + 0.5 * w
    pred_ctr = deltas[:, 0] * w + ctr
    pred_w = jnp.exp(jnp.clip(deltas[:, 1], -10.0, 10.0)) * w
    s = jnp.clip(pred_ctr - 0.5 * pred_w, 0.0, L - 1.0)
    e = jnp.clip(pred_ctr + 0.5 * pred_w, 0.0, L - 1.0)
    return scores, s, e


def _nms_body(scr, ssr, eer, scc, ssc, eec, osc, oss, oee, sup_ref):
    osc[...] = jnp.zeros_like(osc)
    oss[...] = jnp.zeros_like(oss)
    oee[...] = jnp.zeros_like(oee)
    sup_ref[...] = jnp.zeros_like(sup_ref)
    s_row = ssr[0:1, :]
    e_row = eer[0:1, :]
    lens = e_row - s_row + 1.0
    lane = jax.lax.broadcasted_iota(jnp.int32, (1, NPAD), 1)

    def cond_fn(st):
        i, cursor = st
        return jnp.logical_and(i < PRE_N, cursor < POST_N)

    def body_fn(st):
        i, cursor = st
        sup = sup_ref[0:1, :] != 0.0
        sup_i = jnp.any(jnp.logical_and(sup, lane == i))
        keep_i = jnp.logical_not(sup_i)

        @pl.when(keep_i)
        def _():
            si = ssc[pl.ds(i, 1), :]
            ei = eec[pl.ds(i, 1), :]
            li = ei - si + 1.0
            inter = jnp.maximum(0.0, jnp.minimum(ei, e_row) - jnp.maximum(si, s_row) + 1.0)
            iou = inter / (li + lens - inter)
            newly = (iou > NMS_THRESH) & (lane > i)
            sup_ref[0:1, :] = jnp.where(sup | newly, 1.0, 0.0)
            osc[pl.ds(cursor, 1), :] = scc[pl.ds(i, 1), :]
            oss[pl.ds(cursor, 1), :] = si
            oee[pl.ds(cursor, 1), :] = ei

        return i + 1, cursor + keep_i.astype(jnp.int32)

    jax.lax.while_loop(cond_fn, body_fn, (jnp.int32(0), jnp.int32(0)))


def _nms_topk(sc, ss, ee):
    pad = NPAD - PRE_N
    neg = jnp.full((pad,), -3.0e9, jnp.float32)
    scr = jnp.concatenate([sc, jnp.zeros((pad,), jnp.float32)]).reshape(1, NPAD)
    ssr = jnp.concatenate([ss, neg]).reshape(1, NPAD)
    eer = jnp.concatenate([ee, neg]).reshape(1, NPAD)
    scc = scr.reshape(NPAD, 1)
    ssc = ssr.reshape(NPAD, 1)
    eec = eer.reshape(NPAD, 1)
    out = pl.pallas_call(
        _nms_body,
        out_shape=[jax.ShapeDtypeStruct((OUT_R, 1), jnp.float32)] * 3,
        scratch_shapes=[pltpu.VMEM((1, NPAD), jnp.float32)],
    )(scr, ssr, eer, scc, ssc, eec)
    osc, oss, oee = out
    return jnp.stack([osc[:POST_N, 0], oss[:POST_N, 0], oee[:POST_N, 0]], axis=-1)


def kernel(sequence, params):
    scores, s, e = _backbone(sequence, params)
    topv, topi = jax.lax.top_k(scores, PRE_N)
    ss = s[topi]
    ee = e[topi]
    out3 = _nms_topk(topv, ss, ee)
    return out3[:, None, :]
